# Initial kernel scaffold; baseline (speedup 1.0000x reference)
import functools

import jax
import jax.numpy as jnp
from jax import lax
from jax.experimental import pallas as pl
from jax.experimental.pallas import tpu as pltpu
from jax.experimental.pallas import tpu_sc as plsc

N_TIMES = 288
BATCH = 16384
D_LINK = 32
D_TIME = 8
D_DRIVER = 16

NUM_WORKERS = 32
B_PER_W = BATCH // NUM_WORKERS
CHUNK = 128
N_CHUNKS = B_PER_W // CHUNK
LANES = 16


def _sc_gather_kernel(link_idx, time_idx, driver_idx,
                      link_table, time_table, driver_table, cross_table,
                      le_out, te_out, de_out, cv_out,
                      lidx_v, tidx_v, didx_v, cidx_v,
                      le_v, te_v, de_v, cv_v, sem):
    wid = lax.axis_index("s") * 2 + lax.axis_index("c")
    base = wid * B_PER_W

    pltpu.sync_copy(link_idx.at[pl.ds(base, B_PER_W)], lidx_v)
    pltpu.sync_copy(time_idx.at[pl.ds(base, B_PER_W)], tidx_v)
    pltpu.sync_copy(driver_idx.at[pl.ds(base, B_PER_W)], didx_v)

    for i in range(B_PER_W // LANES):
        sl = pl.ds(i * LANES, LANES)
        cidx_v[sl] = lidx_v[sl] * N_TIMES + tidx_v[sl]

    copies = []
    for j in range(N_CHUNKS):
        isl = pl.ds(j * CHUNK, CHUNK)
        copies.append(pltpu.async_copy(
            link_table.at[lidx_v.at[isl]], le_v.at[isl], sem))
        copies.append(pltpu.async_copy(
            time_table.at[tidx_v.at[isl]], te_v.at[isl], sem))
        copies.append(pltpu.async_copy(
            driver_table.at[didx_v.at[isl]], de_v.at[isl], sem))
        copies.append(pltpu.async_copy(
            cross_table.at[cidx_v.at[isl]], cv_v.at[isl], sem))
    for c in copies:
        c.wait()

    pltpu.sync_copy(le_v, le_out.at[pl.ds(base, B_PER_W)])
    pltpu.sync_copy(te_v, te_out.at[pl.ds(base, B_PER_W)])
    pltpu.sync_copy(de_v, de_out.at[pl.ds(base, B_PER_W)])
    pltpu.sync_copy(cv_v, cv_out.at[pl.ds(base, B_PER_W)])


_sc_gather = pl.kernel(
    _sc_gather_kernel,
    out_type=(
        jax.ShapeDtypeStruct((BATCH, D_LINK), jnp.float32),
        jax.ShapeDtypeStruct((BATCH, D_TIME), jnp.float32),
        jax.ShapeDtypeStruct((BATCH, D_DRIVER), jnp.float32),
        jax.ShapeDtypeStruct((BATCH, 1), jnp.float32),
    ),
    mesh=plsc.VectorSubcoreMesh(core_axis_name="c", subcore_axis_name="s"),
    scratch_types=[
        pltpu.VMEM((B_PER_W,), jnp.int32),
        pltpu.VMEM((B_PER_W,), jnp.int32),
        pltpu.VMEM((B_PER_W,), jnp.int32),
        pltpu.VMEM((B_PER_W,), jnp.int32),
        pltpu.VMEM((B_PER_W, D_LINK), jnp.float32),
        pltpu.VMEM((B_PER_W, D_TIME), jnp.float32),
        pltpu.VMEM((B_PER_W, D_DRIVER), jnp.float32),
        pltpu.VMEM((B_PER_W, 1), jnp.float32),
        pltpu.SemaphoreType.DMA,
    ],
)


BB = 2048


def _tc_mlp_kernel(le, te, de, cv, w1l, w1t, w1d, b1, w2, b2, w3, b3, out):
    h = jnp.dot(le[...], w1l[...], preferred_element_type=jnp.float32)
    h += jnp.dot(te[...], w1t[...], preferred_element_type=jnp.float32)
    h += jnp.dot(de[...], w1d[...], preferred_element_type=jnp.float32)
    h = jnp.maximum(h + b1[...], 0.0)
    h2 = jnp.dot(h, w2[...], preferred_element_type=jnp.float32)
    h2 = jnp.maximum(h2 + b2[...], 0.0)
    y = jnp.dot(h2, w3[...], preferred_element_type=jnp.float32)
    out[...] = y + b3[...] + cv[...]


def _tc_mlp(le, te, de, cv, w1l, w1t, w1d, b1, w2, b2, w3, b3):
    grid = (BATCH // BB,)
    row_spec = lambda d: pl.BlockSpec((BB, d), lambda i: (i, 0))
    full = lambda a: pl.BlockSpec(a.shape, lambda i: (0,) * a.ndim)
    return pl.pallas_call(
        _tc_mlp_kernel,
        grid=grid,
        in_specs=[
            row_spec(D_LINK), row_spec(D_TIME), row_spec(D_DRIVER),
            row_spec(1),
            full(w1l), full(w1t), full(w1d), full(b1),
            full(w2), full(b2), full(w3), full(b3),
        ],
        out_specs=pl.BlockSpec((BB, 1), lambda i: (i, 0)),
        out_shape=jax.ShapeDtypeStruct((BATCH, 1), jnp.float32),
    )(le, te, de, cv, w1l, w1t, w1d, b1, w2, b2, w3, b3)


def kernel(link_idx, time_idx, driver_idx, link_table, time_table,
           driver_table, cross_table, W1, b1, W2, b2, W3, b3):
    le, te, de, cv = _sc_gather(
        link_idx.astype(jnp.int32), time_idx.astype(jnp.int32),
        driver_idx.astype(jnp.int32),
        link_table, time_table, driver_table, cross_table)
    w1l = W1[:D_LINK]
    w1t = W1[D_LINK:D_LINK + D_TIME]
    w1d = W1[D_LINK + D_TIME:]
    y = _tc_mlp(le, te, de, cv,
                w1l, w1t, w1d, b1.reshape(1, -1),
                W2, b2.reshape(1, -1), W3, b3.reshape(1, 1))
    return y.reshape(BATCH)


# speedup vs baseline: 1.6812x; 1.6812x over previous
import functools

import jax
import jax.numpy as jnp
from jax import lax
from jax.experimental import pallas as pl
from jax.experimental.pallas import tpu as pltpu
from jax.experimental.pallas import tpu_sc as plsc

N_TIMES = 288
BATCH = 16384
D_LINK = 32
D_TIME = 8
D_DRIVER = 16

NUM_WORKERS = 32
B_PER_W = BATCH // NUM_WORKERS
CHUNK = 128
N_CHUNKS = B_PER_W // CHUNK
LANES = 16


def _sc_gather_kernel(link_idx, time_idx, driver_idx,
                      link_table, time_table, driver_table, cross_flat,
                      le_out, te_out, de_out, cv_out,
                      lidx_v, tidx_v, didx_v, cidx_v,
                      le_v, te_v, de_v, cv_v, sem):
    wid = lax.axis_index("s") * 2 + lax.axis_index("c")
    base = wid * B_PER_W

    pltpu.sync_copy(link_idx.at[pl.ds(base, B_PER_W)], lidx_v)
    pltpu.sync_copy(time_idx.at[pl.ds(base, B_PER_W)], tidx_v)
    pltpu.sync_copy(driver_idx.at[pl.ds(base, B_PER_W)], didx_v)

    for i in range(B_PER_W // LANES):
        sl = pl.ds(i * LANES, LANES)
        cidx_v[sl] = lidx_v[sl] * N_TIMES + tidx_v[sl]

    copies = []
    for j in range(N_CHUNKS):
        isl = pl.ds(j * CHUNK, CHUNK)
        copies.append(pltpu.async_copy(
            link_table.at[lidx_v.at[isl]], le_v.at[isl], sem))
        copies.append(pltpu.async_copy(
            time_table.at[tidx_v.at[isl]], te_v.at[isl], sem))
        copies.append(pltpu.async_copy(
            driver_table.at[didx_v.at[isl]], de_v.at[isl], sem))
        copies.append(pltpu.async_copy(
            cross_flat.at[cidx_v.at[isl]], cv_v.at[isl], sem))
    for c in copies:
        c.wait()

    pltpu.sync_copy(le_v, le_out.at[pl.ds(base, B_PER_W)])
    pltpu.sync_copy(te_v, te_out.at[pl.ds(base, B_PER_W)])
    pltpu.sync_copy(de_v, de_out.at[pl.ds(base, B_PER_W)])
    pltpu.sync_copy(cv_v, cv_out.at[pl.ds(base, B_PER_W)])


_sc_gather = pl.kernel(
    _sc_gather_kernel,
    out_type=(
        jax.ShapeDtypeStruct((BATCH, D_LINK), jnp.float32),
        jax.ShapeDtypeStruct((BATCH, D_TIME), jnp.float32),
        jax.ShapeDtypeStruct((BATCH, D_DRIVER), jnp.float32),
        jax.ShapeDtypeStruct((BATCH,), jnp.float32),
    ),
    mesh=plsc.VectorSubcoreMesh(core_axis_name="c", subcore_axis_name="s"),
    scratch_types=[
        pltpu.VMEM((B_PER_W,), jnp.int32),
        pltpu.VMEM((B_PER_W,), jnp.int32),
        pltpu.VMEM((B_PER_W,), jnp.int32),
        pltpu.VMEM((B_PER_W,), jnp.int32),
        pltpu.VMEM((B_PER_W, D_LINK), jnp.float32),
        pltpu.VMEM((B_PER_W, D_TIME), jnp.float32),
        pltpu.VMEM((B_PER_W, D_DRIVER), jnp.float32),
        pltpu.VMEM((B_PER_W,), jnp.float32),
        pltpu.SemaphoreType.DMA,
    ],
    compiler_params=pltpu.CompilerParams(use_tc_tiling_on_sc=False),
)


BB = 2048


def _tc_mlp_kernel(le, te, de, cv, w1l, w1t, w1d, b1, w2, b2, w3, b3, out):
    h = jnp.dot(le[...], w1l[...], preferred_element_type=jnp.float32)
    h += jnp.dot(te[...], w1t[...], preferred_element_type=jnp.float32)
    h += jnp.dot(de[...], w1d[...], preferred_element_type=jnp.float32)
    h = jnp.maximum(h + b1[...], 0.0)
    h2 = jnp.dot(h, w2[...], preferred_element_type=jnp.float32)
    h2 = jnp.maximum(h2 + b2[...], 0.0)
    y = jnp.dot(h2, w3[...], preferred_element_type=jnp.float32)
    out[...] = y + b3[...] + cv[...]


def _tc_mlp(le, te, de, cv, w1l, w1t, w1d, b1, w2, b2, w3, b3):
    grid = (BATCH // BB,)
    row_spec = lambda d: pl.BlockSpec((BB, d), lambda i: (i, 0))
    full = lambda a: pl.BlockSpec(a.shape, lambda i: (0,) * a.ndim)
    return pl.pallas_call(
        _tc_mlp_kernel,
        grid=grid,
        in_specs=[
            row_spec(D_LINK), row_spec(D_TIME), row_spec(D_DRIVER),
            row_spec(1),
            full(w1l), full(w1t), full(w1d), full(b1),
            full(w2), full(b2), full(w3), full(b3),
        ],
        out_specs=pl.BlockSpec((BB, 1), lambda i: (i, 0)),
        out_shape=jax.ShapeDtypeStruct((BATCH, 1), jnp.float32),
    )(le, te, de, cv, w1l, w1t, w1d, b1, w2, b2, w3, b3)


def kernel(link_idx, time_idx, driver_idx, link_table, time_table,
           driver_table, cross_table, W1, b1, W2, b2, W3, b3):
    le, te, de, cv = _sc_gather(
        link_idx.astype(jnp.int32), time_idx.astype(jnp.int32),
        driver_idx.astype(jnp.int32),
        link_table, time_table, driver_table, cross_table.reshape(-1))
    w1l = W1[:D_LINK]
    w1t = W1[D_LINK:D_LINK + D_TIME]
    w1d = W1[D_LINK + D_TIME:]
    y = _tc_mlp(le, te, de, cv.reshape(BATCH, 1),
                w1l, w1t, w1d, b1.reshape(1, -1),
                W2, b2.reshape(1, -1), W3, b3.reshape(1, 1))
    return y.reshape(BATCH)


# speedup vs baseline: 1.9848x; 1.1806x over previous
import functools

import jax
import jax.numpy as jnp
from jax import lax
from jax.experimental import pallas as pl
from jax.experimental.pallas import tpu as pltpu
from jax.experimental.pallas import tpu_sc as plsc

N_LINKS = 100000
N_TIMES = 288
BATCH = 16384
D_LINK = 32
D_TIME = 8
D_DRIVER = 16
DP = 128

NUM_WORKERS = 32
B_PER_W = BATCH // NUM_WORKERS
CHUNK = 128
N_CHUNKS = B_PER_W // CHUNK
LANES = 16

RC = 2048
RG = (N_LINKS + RC - 1) // RC


def _repack_kernel(ltT, dtT, ttT, out):
    i = pl.program_id(0)
    l = ltT[...].T
    d = dtT[...].T
    z = jnp.zeros((RC, DP - D_LINK - D_DRIVER), jnp.float32)
    out[...] = jnp.concatenate([l, d, z], axis=1)

    @pl.when(i == 0)
    def _():
        out[0:N_TIMES, D_LINK + D_DRIVER:D_LINK + D_DRIVER + D_TIME] = (
            ttT[...].T)


def _tc_repack(ltT, dtT, ttT):
    return pl.pallas_call(
        _repack_kernel,
        grid=(RG,),
        in_specs=[
            pl.BlockSpec((D_LINK, RC), lambda i: (0, i)),
            pl.BlockSpec((D_DRIVER, RC), lambda i: (0, i)),
            pl.BlockSpec((D_TIME, N_TIMES), lambda i: (0, 0)),
        ],
        out_specs=pl.BlockSpec((RC, DP), lambda i: (i, 0)),
        out_shape=jax.ShapeDtypeStruct((N_LINKS, DP), jnp.float32),
    )(ltT, dtT, ttT)


def _sc_gather_kernel(link_idx, time_idx, driver_idx, bigP, cross_flat,
                      gl_out, gd_out, gt_out, cv_out,
                      lidx_v, tidx_v, didx_v, cidx_v, gbuf, cv_v, sem, csem):
    wid = lax.axis_index("s") * 2 + lax.axis_index("c")
    base = wid * B_PER_W

    pltpu.sync_copy(link_idx.at[pl.ds(base, B_PER_W)], lidx_v)
    pltpu.sync_copy(time_idx.at[pl.ds(base, B_PER_W)], tidx_v)
    pltpu.sync_copy(driver_idx.at[pl.ds(base, B_PER_W)], didx_v)

    for i in range(B_PER_W // LANES):
        sl = pl.ds(i * LANES, LANES)
        cidx_v[sl] = lidx_v[sl] * N_TIMES + tidx_v[sl]

    ccopies = []
    for j in range(N_CHUNKS):
        isl = pl.ds(j * CHUNK, CHUNK)
        ccopies.append(pltpu.async_copy(
            cross_flat.at[cidx_v.at[isl]], cv_v.at[isl], csem))

    for idx_v, out in ((lidx_v, gl_out), (didx_v, gd_out), (tidx_v, gt_out)):
        copies = []
        for j in range(N_CHUNKS):
            isl = pl.ds(j * CHUNK, CHUNK)
            copies.append(pltpu.async_copy(
                bigP.at[idx_v.at[isl]], gbuf.at[isl], sem))
        for c in copies:
            c.wait()
        pltpu.sync_copy(gbuf, out.at[pl.ds(base, B_PER_W)])

    for c in ccopies:
        c.wait()
    pltpu.sync_copy(cv_v, cv_out.at[pl.ds(base, B_PER_W)])


_sc_gather = pl.kernel(
    _sc_gather_kernel,
    out_type=(
        jax.ShapeDtypeStruct((BATCH, DP), jnp.float32),
        jax.ShapeDtypeStruct((BATCH, DP), jnp.float32),
        jax.ShapeDtypeStruct((BATCH, DP), jnp.float32),
        jax.ShapeDtypeStruct((BATCH,), jnp.float32),
    ),
    mesh=plsc.VectorSubcoreMesh(core_axis_name="c", subcore_axis_name="s"),
    scratch_types=[
        pltpu.VMEM((B_PER_W,), jnp.int32),
        pltpu.VMEM((B_PER_W,), jnp.int32),
        pltpu.VMEM((B_PER_W,), jnp.int32),
        pltpu.VMEM((B_PER_W,), jnp.int32),
        pltpu.VMEM((B_PER_W, DP), jnp.float32),
        pltpu.VMEM((B_PER_W,), jnp.float32),
        pltpu.SemaphoreType.DMA,
        pltpu.SemaphoreType.DMA,
    ],
    compiler_params=pltpu.CompilerParams(use_tc_tiling_on_sc=True),
)


BB = 2048


def _tc_mlp_kernel(gl, gd, gt, cv, w1a, w1b, w1c, b1, w2, b2, w3, b3, out):
    h = jnp.dot(gl[...], w1a[...], preferred_element_type=jnp.float32)
    h += jnp.dot(gd[...], w1b[...], preferred_element_type=jnp.float32)
    h += jnp.dot(gt[...], w1c[...], preferred_element_type=jnp.float32)
    h = jnp.maximum(h + b1[...], 0.0)
    h2 = jnp.dot(h, w2[...], preferred_element_type=jnp.float32)
    h2 = jnp.maximum(h2 + b2[...], 0.0)
    y = jnp.sum(h2 * w3[...], axis=1)
    out[...] = y + b3[...] + cv[...]


def _tc_mlp(gl, gd, gt, cv, w1a, w1b, w1c, b1, w2, b2, w3, b3):
    grid = (BATCH // BB,)
    row = pl.BlockSpec((BB, DP), lambda i: (i, 0))
    vec = pl.BlockSpec((BB,), lambda i: (i,))
    full = lambda a: pl.BlockSpec(a.shape, lambda i: (0,) * a.ndim)
    return pl.pallas_call(
        _tc_mlp_kernel,
        grid=grid,
        in_specs=[
            row, row, row, vec,
            full(w1a), full(w1b), full(w1c), full(b1),
            full(w2), full(b2), full(w3), full(b3),
        ],
        out_specs=vec,
        out_shape=jax.ShapeDtypeStruct((BATCH,), jnp.float32),
    )(gl, gd, gt, cv, w1a, w1b, w1c, b1, w2, b2, w3, b3)


def kernel(link_idx, time_idx, driver_idx, link_table, time_table,
           driver_table, cross_table, W1, b1, W2, b2, W3, b3):
    bigP = _tc_repack(link_table.T, driver_table.T, time_table.T)
    gl, gd, gt, cv = _sc_gather(
        link_idx.astype(jnp.int32), time_idx.astype(jnp.int32),
        driver_idx.astype(jnp.int32), bigP, cross_table.reshape(-1))
    z = jnp.zeros((DP, 128), jnp.float32)
    w1a = z.at[0:D_LINK].set(W1[0:D_LINK])
    w1b = z.at[D_LINK:D_LINK + D_DRIVER].set(W1[D_LINK + D_TIME:])
    w1c = z.at[D_LINK + D_DRIVER:D_LINK + D_DRIVER + D_TIME].set(
        W1[D_LINK:D_LINK + D_TIME])
    y = _tc_mlp(gl, gd, gt, cv, w1a, w1b, w1c, b1.reshape(1, -1),
                W2, b2.reshape(1, -1), W3.reshape(1, -1), b3)
    return y


# speedup vs baseline: 2.0784x; 1.0472x over previous
import functools

import jax
import jax.numpy as jnp
from jax import lax
from jax.experimental import pallas as pl
from jax.experimental.pallas import tpu as pltpu
from jax.experimental.pallas import tpu_sc as plsc

N_LINKS = 100000
N_TIMES = 288
BATCH = 16384
D_LINK = 32
D_TIME = 8
D_DRIVER = 16
DP = 128

NUM_WORKERS = 32
B_PER_W = BATCH // NUM_WORKERS
CHUNK = 128
N_CHUNKS = B_PER_W // CHUNK
LANES = 16

RC = 2048
RG = (N_LINKS + RC - 1) // RC


def _repack_kernel(ltT, dtT, ttT, out):
    i = pl.program_id(0)
    l = ltT[...].T
    d = dtT[...].T
    z = jnp.zeros((RC, DP - D_LINK - D_DRIVER), jnp.float32)
    out[...] = jnp.concatenate([l, d, z], axis=1)

    @pl.when(i == 0)
    def _():
        out[0:N_TIMES, D_LINK + D_DRIVER:D_LINK + D_DRIVER + D_TIME] = (
            ttT[...].T)


def _tc_repack(ltT, dtT, ttT):
    return pl.pallas_call(
        _repack_kernel,
        grid=(RG,),
        in_specs=[
            pl.BlockSpec((D_LINK, RC), lambda i: (0, i)),
            pl.BlockSpec((D_DRIVER, RC), lambda i: (0, i)),
            pl.BlockSpec((D_TIME, N_TIMES), lambda i: (0, 0)),
        ],
        out_specs=pl.BlockSpec((RC, DP), lambda i: (i, 0)),
        out_shape=jax.ShapeDtypeStruct((N_LINKS, DP), jnp.float32),
    )(ltT, dtT, ttT)


def _sc_gather_kernel(link_idx, time_idx, driver_idx, bigP, cross_flat,
                      gl_out, gd_out, gt_out, cv_out,
                      lidx_v, tidx_v, didx_v, cidx_v,
                      glb0, gdb0, gtb0, glb1, gdb1, gtb1,
                      cv_v, sem, csem):
    wid = lax.axis_index("s") * 2 + lax.axis_index("c")
    base = wid * B_PER_W

    pltpu.sync_copy(link_idx.at[pl.ds(base, B_PER_W)], lidx_v)
    pltpu.sync_copy(time_idx.at[pl.ds(base, B_PER_W)], tidx_v)
    pltpu.sync_copy(driver_idx.at[pl.ds(base, B_PER_W)], didx_v)

    for i in range(B_PER_W // LANES):
        sl = pl.ds(i * LANES, LANES)
        cidx_v[sl] = lidx_v[sl] * N_TIMES + tidx_v[sl]

    ccopies = []
    for j in range(N_CHUNKS):
        isl = pl.ds(j * CHUNK, CHUNK)
        ccopies.append(pltpu.async_copy(
            cross_flat.at[cidx_v.at[isl]], cv_v.at[isl], csem))

    sets = ((glb0, gdb0, gtb0), (glb1, gdb1, gtb1))
    outs = (gl_out, gd_out, gt_out)

    def fire(q):
        isl = pl.ds(q * CHUNK, CHUNK)
        return [
            pltpu.async_copy(bigP.at[idx_v.at[isl]], buf, sem)
            for idx_v, buf in zip((lidx_v, didx_v, tidx_v), sets[q % 2])
        ]

    pend = fire(0)
    for q in range(N_CHUNKS):
        for c in pend:
            c.wait()
        if q + 1 < N_CHUNKS:
            pend = fire(q + 1)
        for buf, out in zip(sets[q % 2], outs):
            pltpu.sync_copy(buf, out.at[pl.ds(base + q * CHUNK, CHUNK)])

    for c in ccopies:
        c.wait()
    pltpu.sync_copy(cv_v, cv_out.at[pl.ds(base, B_PER_W)])


_sc_gather = pl.kernel(
    _sc_gather_kernel,
    out_type=(
        jax.ShapeDtypeStruct((BATCH, DP), jnp.float32),
        jax.ShapeDtypeStruct((BATCH, DP), jnp.float32),
        jax.ShapeDtypeStruct((BATCH, DP), jnp.float32),
        jax.ShapeDtypeStruct((BATCH,), jnp.float32),
    ),
    mesh=plsc.VectorSubcoreMesh(core_axis_name="c", subcore_axis_name="s"),
    scratch_types=[
        pltpu.VMEM((B_PER_W,), jnp.int32),
        pltpu.VMEM((B_PER_W,), jnp.int32),
        pltpu.VMEM((B_PER_W,), jnp.int32),
        pltpu.VMEM((B_PER_W,), jnp.int32),
        pltpu.VMEM((CHUNK, DP), jnp.float32),
        pltpu.VMEM((CHUNK, DP), jnp.float32),
        pltpu.VMEM((CHUNK, DP), jnp.float32),
        pltpu.VMEM((CHUNK, DP), jnp.float32),
        pltpu.VMEM((CHUNK, DP), jnp.float32),
        pltpu.VMEM((CHUNK, DP), jnp.float32),
        pltpu.VMEM((B_PER_W,), jnp.float32),
        pltpu.SemaphoreType.DMA,
        pltpu.SemaphoreType.DMA,
    ],
    compiler_params=pltpu.CompilerParams(use_tc_tiling_on_sc=True),
)


BB = 4096


def _tc_mlp_kernel(gl, gd, gt, cv, w1a, w1b, w1c, b1, w2, b2, w3, b3, out):
    h = jnp.dot(gl[...], w1a[...], preferred_element_type=jnp.float32)
    h += jnp.dot(gd[...], w1b[...], preferred_element_type=jnp.float32)
    h += jnp.dot(gt[...], w1c[...], preferred_element_type=jnp.float32)
    h = jnp.maximum(h + b1[...], 0.0)
    h2 = jnp.dot(h, w2[...], preferred_element_type=jnp.float32)
    h2 = jnp.maximum(h2 + b2[...], 0.0)
    y = jnp.sum(h2 * w3[...], axis=1)
    out[...] = y + b3[...] + cv[...]


def _tc_mlp(gl, gd, gt, cv, w1a, w1b, w1c, b1, w2, b2, w3, b3):
    grid = (BATCH // BB,)
    row = pl.BlockSpec((BB, DP), lambda i: (i, 0))
    vec = pl.BlockSpec((BB,), lambda i: (i,))
    full = lambda a: pl.BlockSpec(a.shape, lambda i: (0,) * a.ndim)
    return pl.pallas_call(
        _tc_mlp_kernel,
        grid=grid,
        in_specs=[
            row, row, row, vec,
            full(w1a), full(w1b), full(w1c), full(b1),
            full(w2), full(b2), full(w3), full(b3),
        ],
        out_specs=vec,
        out_shape=jax.ShapeDtypeStruct((BATCH,), jnp.float32),
    )(gl, gd, gt, cv, w1a, w1b, w1c, b1, w2, b2, w3, b3)


def kernel(link_idx, time_idx, driver_idx, link_table, time_table,
           driver_table, cross_table, W1, b1, W2, b2, W3, b3):
    bigP = _tc_repack(link_table.T, driver_table.T, time_table.T)
    gl, gd, gt, cv = _sc_gather(
        link_idx.astype(jnp.int32), time_idx.astype(jnp.int32),
        driver_idx.astype(jnp.int32), bigP, cross_table.reshape(-1))
    z = jnp.zeros((DP, 128), jnp.float32)
    w1a = z.at[0:D_LINK].set(W1[0:D_LINK])
    w1b = z.at[D_LINK:D_LINK + D_DRIVER].set(W1[D_LINK + D_TIME:])
    w1c = z.at[D_LINK + D_DRIVER:D_LINK + D_DRIVER + D_TIME].set(
        W1[D_LINK:D_LINK + D_TIME])
    y = _tc_mlp(gl, gd, gt, cv, w1a, w1b, w1c, b1.reshape(1, -1),
                W2, b2.reshape(1, -1), W3.reshape(1, -1), b3)
    return y


# speedup vs baseline: 2.3723x; 1.1414x over previous
import functools

import jax
import jax.numpy as jnp
from jax import lax
from jax.experimental import pallas as pl
from jax.experimental.pallas import tpu as pltpu
from jax.experimental.pallas import tpu_sc as plsc

N_LINKS = 100000
N_TIMES = 288
BATCH = 16384
D_LINK = 32
D_TIME = 8
D_DRIVER = 16
DP = 128

NUM_WORKERS = 32
B_PER_W = BATCH // NUM_WORKERS
CHUNK = 128
N_CHUNKS = B_PER_W // CHUNK
LANES = 16

RC = 4096
RG = (N_LINKS + RC - 1) // RC


def _repack_kernel(ltT, dtT, ttT, out):
    i = pl.program_id(0)
    dims = (((0,), (0,)), ((), ()))
    p1 = jnp.eye(D_LINK, DP, 0, dtype=jnp.float32)
    p2 = jnp.eye(D_DRIVER, DP, D_LINK, dtype=jnp.float32)
    out[...] = (
        jax.lax.dot_general(ltT[...], p1, dims,
                            preferred_element_type=jnp.float32)
        + jax.lax.dot_general(dtT[...], p2, dims,
                              preferred_element_type=jnp.float32))

    @pl.when(i == 0)
    def _():
        p3 = jnp.eye(D_TIME, D_TIME, 0, dtype=jnp.float32)
        out[0:N_TIMES, D_LINK + D_DRIVER:D_LINK + D_DRIVER + D_TIME] = (
            jax.lax.dot_general(ttT[...], p3, dims,
                                preferred_element_type=jnp.float32))


def _tc_repack(ltT, dtT, ttT):
    return pl.pallas_call(
        _repack_kernel,
        grid=(RG,),
        in_specs=[
            pl.BlockSpec((D_LINK, RC), lambda i: (0, i)),
            pl.BlockSpec((D_DRIVER, RC), lambda i: (0, i)),
            pl.BlockSpec((D_TIME, N_TIMES), lambda i: (0, 0)),
        ],
        out_specs=pl.BlockSpec((RC, DP), lambda i: (i, 0)),
        out_shape=jax.ShapeDtypeStruct((N_LINKS, DP), jnp.float32),
    )(ltT, dtT, ttT)


def _sc_gather_kernel(link_idx, time_idx, driver_idx, bigP, cross_flat,
                      gl_out, gd_out, gt_out, cv_out,
                      lidx_v, tidx_v, didx_v, cidx_v,
                      glb0, gdb0, gtb0, glb1, gdb1, gtb1,
                      cv_v, sem, csem):
    wid = lax.axis_index("s") * 2 + lax.axis_index("c")
    base = wid * B_PER_W

    pltpu.sync_copy(link_idx.at[pl.ds(base, B_PER_W)], lidx_v)
    pltpu.sync_copy(time_idx.at[pl.ds(base, B_PER_W)], tidx_v)
    pltpu.sync_copy(driver_idx.at[pl.ds(base, B_PER_W)], didx_v)

    for i in range(B_PER_W // LANES):
        sl = pl.ds(i * LANES, LANES)
        cidx_v[sl] = lidx_v[sl] * N_TIMES + tidx_v[sl]

    ccopies = []
    for j in range(N_CHUNKS):
        isl = pl.ds(j * CHUNK, CHUNK)
        ccopies.append(pltpu.async_copy(
            cross_flat.at[cidx_v.at[isl]], cv_v.at[isl], csem))

    sets = ((glb0, gdb0, gtb0), (glb1, gdb1, gtb1))
    outs = (gl_out, gd_out, gt_out)

    def fire(q):
        isl = pl.ds(q * CHUNK, CHUNK)
        return [
            pltpu.async_copy(bigP.at[idx_v.at[isl]], buf, sem)
            for idx_v, buf in zip((lidx_v, didx_v, tidx_v), sets[q % 2])
        ]

    pend = fire(0)
    for q in range(N_CHUNKS):
        for c in pend:
            c.wait()
        if q + 1 < N_CHUNKS:
            pend = fire(q + 1)
        for buf, out in zip(sets[q % 2], outs):
            pltpu.sync_copy(buf, out.at[pl.ds(base + q * CHUNK, CHUNK)])

    for c in ccopies:
        c.wait()
    pltpu.sync_copy(cv_v, cv_out.at[pl.ds(base, B_PER_W)])


_sc_gather = pl.kernel(
    _sc_gather_kernel,
    out_type=(
        jax.ShapeDtypeStruct((BATCH, DP), jnp.float32),
        jax.ShapeDtypeStruct((BATCH, DP), jnp.float32),
        jax.ShapeDtypeStruct((BATCH, DP), jnp.float32),
        jax.ShapeDtypeStruct((BATCH,), jnp.float32),
    ),
    mesh=plsc.VectorSubcoreMesh(core_axis_name="c", subcore_axis_name="s"),
    scratch_types=[
        pltpu.VMEM((B_PER_W,), jnp.int32),
        pltpu.VMEM((B_PER_W,), jnp.int32),
        pltpu.VMEM((B_PER_W,), jnp.int32),
        pltpu.VMEM((B_PER_W,), jnp.int32),
        pltpu.VMEM((CHUNK, DP), jnp.float32),
        pltpu.VMEM((CHUNK, DP), jnp.float32),
        pltpu.VMEM((CHUNK, DP), jnp.float32),
        pltpu.VMEM((CHUNK, DP), jnp.float32),
        pltpu.VMEM((CHUNK, DP), jnp.float32),
        pltpu.VMEM((CHUNK, DP), jnp.float32),
        pltpu.VMEM((B_PER_W,), jnp.float32),
        pltpu.SemaphoreType.DMA,
        pltpu.SemaphoreType.DMA,
    ],
    compiler_params=pltpu.CompilerParams(use_tc_tiling_on_sc=True),
)


BB = 4096


def _tc_mlp_kernel(gl, gd, gt, cv, w1a, w1b, w1c, b1, w2, b2, w3, b3, out):
    h = jnp.dot(gl[...], w1a[...], preferred_element_type=jnp.float32)
    h += jnp.dot(gd[...], w1b[...], preferred_element_type=jnp.float32)
    h += jnp.dot(gt[...], w1c[...], preferred_element_type=jnp.float32)
    h = jnp.maximum(h + b1[...], 0.0)
    h2 = jnp.dot(h, w2[...], preferred_element_type=jnp.float32)
    h2 = jnp.maximum(h2 + b2[...], 0.0)
    y = jnp.sum(h2 * w3[...], axis=1)
    out[...] = y + b3[...] + cv[...]


def _tc_mlp(gl, gd, gt, cv, w1a, w1b, w1c, b1, w2, b2, w3, b3):
    grid = (BATCH // BB,)
    row = pl.BlockSpec((BB, DP), lambda i: (i, 0))
    vec = pl.BlockSpec((BB,), lambda i: (i,))
    full = lambda a: pl.BlockSpec(a.shape, lambda i: (0,) * a.ndim)
    return pl.pallas_call(
        _tc_mlp_kernel,
        grid=grid,
        in_specs=[
            row, row, row, vec,
            full(w1a), full(w1b), full(w1c), full(b1),
            full(w2), full(b2), full(w3), full(b3),
        ],
        out_specs=vec,
        out_shape=jax.ShapeDtypeStruct((BATCH,), jnp.float32),
    )(gl, gd, gt, cv, w1a, w1b, w1c, b1, w2, b2, w3, b3)


def kernel(link_idx, time_idx, driver_idx, link_table, time_table,
           driver_table, cross_table, W1, b1, W2, b2, W3, b3):
    bigP = _tc_repack(link_table.T, driver_table.T, time_table.T)
    gl, gd, gt, cv = _sc_gather(
        link_idx.astype(jnp.int32), time_idx.astype(jnp.int32),
        driver_idx.astype(jnp.int32), bigP, cross_table.reshape(-1))
    z = jnp.zeros((DP, 128), jnp.float32)
    w1a = z.at[0:D_LINK].set(W1[0:D_LINK])
    w1b = z.at[D_LINK:D_LINK + D_DRIVER].set(W1[D_LINK + D_TIME:])
    w1c = z.at[D_LINK + D_DRIVER:D_LINK + D_DRIVER + D_TIME].set(
        W1[D_LINK:D_LINK + D_TIME])
    y = _tc_mlp(gl, gd, gt, cv, w1a, w1b, w1c, b1.reshape(1, -1),
                W2, b2.reshape(1, -1), W3.reshape(1, -1), b3)
    return y


# speedup vs baseline: 3.0063x; 1.2673x over previous
import functools

import jax
import jax.numpy as jnp
from jax import lax
from jax.experimental import pallas as pl
from jax.experimental.pallas import tpu as pltpu
from jax.experimental.pallas import tpu_sc as plsc

N_LINKS = 100000
N_TIMES = 288
BATCH = 16384
D_LINK = 32
D_TIME = 8
D_DRIVER = 16
DP = 128

NUM_WORKERS = 32
B_PER_W = BATCH // NUM_WORKERS
CHUNK = 128
N_CHUNKS = B_PER_W // CHUNK
LANES = 16

RC = 8192
RG = (N_LINKS + RC - 1) // RC


def _repack_kernel(ltT, dtT, ttT, out):
    i = pl.program_id(0)
    dims = (((0,), (0,)), ((), ()))
    p1 = jnp.eye(D_LINK, DP, 0, dtype=jnp.float32)
    p2 = jnp.eye(D_DRIVER, DP, D_LINK + D_TIME, dtype=jnp.float32)
    out[...] = (
        jax.lax.dot_general(ltT[...], p1, dims,
                            preferred_element_type=jnp.float32)
        + jax.lax.dot_general(dtT[...], p2, dims,
                              preferred_element_type=jnp.float32))

    @pl.when(i == 0)
    def _():
        p3 = jnp.eye(D_TIME, D_TIME, 0, dtype=jnp.float32)
        out[0:N_TIMES, D_LINK:D_LINK + D_TIME] = (
            jax.lax.dot_general(ttT[...], p3, dims,
                                preferred_element_type=jnp.float32))


def _tc_repack(ltT, dtT, ttT):
    return pl.pallas_call(
        _repack_kernel,
        grid=(RG,),
        in_specs=[
            pl.BlockSpec((D_LINK, RC), lambda i: (0, i)),
            pl.BlockSpec((D_DRIVER, RC), lambda i: (0, i)),
            pl.BlockSpec((D_TIME, N_TIMES), lambda i: (0, 0)),
        ],
        out_specs=pl.BlockSpec((RC, DP), lambda i: (i, 0)),
        out_shape=jax.ShapeDtypeStruct((N_LINKS, DP), jnp.float32),
    )(ltT, dtT, ttT)


def _sc_gather_kernel(link_idx, time_idx, driver_idx, bigP, cross_flat,
                      g_out, cv_out,
                      lidx_v, tidx_v, didx_v, cidx_v,
                      glb0, gdb0, gtb0, glb1, gdb1, gtb1,
                      cv_v, sem, csem):
    wid = lax.axis_index("s") * 2 + lax.axis_index("c")
    base = wid * B_PER_W

    pltpu.sync_copy(link_idx.at[pl.ds(base, B_PER_W)], lidx_v)
    pltpu.sync_copy(time_idx.at[pl.ds(base, B_PER_W)], tidx_v)
    pltpu.sync_copy(driver_idx.at[pl.ds(base, B_PER_W)], didx_v)

    for i in range(B_PER_W // LANES):
        sl = pl.ds(i * LANES, LANES)
        cidx_v[sl] = lidx_v[sl] * N_TIMES + tidx_v[sl]

    ccopies = []
    for j in range(N_CHUNKS):
        isl = pl.ds(j * CHUNK, CHUNK)
        ccopies.append(pltpu.async_copy(
            cross_flat.at[cidx_v.at[isl]], cv_v.at[isl], csem))

    sets = ((glb0, gdb0, gtb0), (glb1, gdb1, gtb1))

    def fire(q):
        isl = pl.ds(q * CHUNK, CHUNK)
        return [
            pltpu.async_copy(bigP.at[idx_v.at[isl]], buf, sem)
            for idx_v, buf in zip((lidx_v, didx_v, tidx_v), sets[q % 2])
        ]

    pend = fire(0)
    for q in range(N_CHUNKS):
        for c in pend:
            c.wait()
        if q + 1 < N_CHUNKS:
            pend = fire(q + 1)
        glbuf, gdbuf, gtbuf = sets[q % 2]
        lane = lax.broadcasted_iota(jnp.int32, (LANES,), 0)

        def merge_row(r, _):
            a = gtbuf[r, pl.ds(D_LINK, LANES)]
            b = gdbuf[r, pl.ds(D_LINK, LANES)]
            glbuf[r, pl.ds(D_LINK, LANES)] = jnp.where(lane < D_TIME, a, b)
            glbuf[r, pl.ds(D_LINK + LANES, LANES)] = (
                gdbuf[r, pl.ds(D_LINK + LANES, LANES)])
            return _

        lax.fori_loop(0, CHUNK, merge_row, None)
        pltpu.sync_copy(glbuf, g_out.at[pl.ds(base + q * CHUNK, CHUNK)])

    for c in ccopies:
        c.wait()
    pltpu.sync_copy(cv_v, cv_out.at[pl.ds(base, B_PER_W)])


_sc_gather = pl.kernel(
    _sc_gather_kernel,
    out_type=(
        jax.ShapeDtypeStruct((BATCH, DP), jnp.float32),
        jax.ShapeDtypeStruct((BATCH,), jnp.float32),
    ),
    mesh=plsc.VectorSubcoreMesh(core_axis_name="c", subcore_axis_name="s"),
    scratch_types=[
        pltpu.VMEM((B_PER_W,), jnp.int32),
        pltpu.VMEM((B_PER_W,), jnp.int32),
        pltpu.VMEM((B_PER_W,), jnp.int32),
        pltpu.VMEM((B_PER_W,), jnp.int32),
        pltpu.VMEM((CHUNK, DP), jnp.float32),
        pltpu.VMEM((CHUNK, DP), jnp.float32),
        pltpu.VMEM((CHUNK, DP), jnp.float32),
        pltpu.VMEM((CHUNK, DP), jnp.float32),
        pltpu.VMEM((CHUNK, DP), jnp.float32),
        pltpu.VMEM((CHUNK, DP), jnp.float32),
        pltpu.VMEM((B_PER_W,), jnp.float32),
        pltpu.SemaphoreType.DMA,
        pltpu.SemaphoreType.DMA,
    ],
    compiler_params=pltpu.CompilerParams(use_tc_tiling_on_sc=True),
)


BB = 4096


def _tc_mlp_kernel(g, cv, w1, b1, w2, b2, w3, b3, out):
    x = g[...][:, 0:D_LINK + D_TIME + D_DRIVER]
    h = jnp.dot(x, w1[...], preferred_element_type=jnp.float32)
    h = jnp.maximum(h + b1[...], 0.0)
    h2 = jnp.dot(h, w2[...], preferred_element_type=jnp.float32)
    h2 = jnp.maximum(h2 + b2[...], 0.0)
    y = jnp.sum(h2 * w3[...], axis=1)
    out[...] = y + b3[...] + cv[...]


def _tc_mlp(g, cv, w1, b1, w2, b2, w3, b3):
    grid = (BATCH // BB,)
    row = pl.BlockSpec((BB, DP), lambda i: (i, 0))
    vec = pl.BlockSpec((BB,), lambda i: (i,))
    full = lambda a: pl.BlockSpec(a.shape, lambda i: (0,) * a.ndim)
    return pl.pallas_call(
        _tc_mlp_kernel,
        grid=grid,
        in_specs=[
            row, vec,
            full(w1), full(b1), full(w2), full(b2), full(w3), full(b3),
        ],
        out_specs=vec,
        out_shape=jax.ShapeDtypeStruct((BATCH,), jnp.float32),
    )(g, cv, w1, b1, w2, b2, w3, b3)


def kernel(link_idx, time_idx, driver_idx, link_table, time_table,
           driver_table, cross_table, W1, b1, W2, b2, W3, b3):
    bigP = _tc_repack(link_table.T, driver_table.T, time_table.T)
    g, cv = _sc_gather(
        link_idx.astype(jnp.int32), time_idx.astype(jnp.int32),
        driver_idx.astype(jnp.int32), bigP, cross_table.reshape(-1))
    y = _tc_mlp(g, cv, W1, b1.reshape(1, -1),
                W2, b2.reshape(1, -1), W3.reshape(1, -1), b3)
    return y
